# baseline (speedup 1.0000x reference)
import functools

import jax
import jax.numpy as jnp
from jax import lax
from jax.experimental import pallas as pl
from jax.experimental.pallas import tpu as pltpu
from jax.experimental.pallas import tpu_sc as plsc

B = 16384
L = 50
V = 1000000
D = 128

NC = 2
NS = 16
NW = NC * NS
IPT = B // NW
K = 8
NCH = IPT // K
ROWS = K * L
LANES = 16
STREAMS = ((0, 128), (128, 128), (256, 128), (384, 16))

_mesh = plsc.VectorSubcoreMesh(core_axis_name="c", subcore_axis_name="s")


@functools.partial(
    pl.kernel,
    out_type=jax.ShapeDtypeStruct((B,), jnp.float32),
    mesh=_mesh,
    compiler_params=pltpu.CompilerParams(
        needs_layout_passes=False,
        disable_bounds_checks=True,
        disable_semaphore_checks=True,
        skip_device_barrier=True,
    ),
    scratch_types=[
        pltpu.VMEM((ROWS,), jnp.int32),
        pltpu.VMEM((ROWS,), jnp.int32),
        pltpu.VMEM((ROWS,), jnp.int32),
        pltpu.VMEM((ROWS,), jnp.int32),
        pltpu.VMEM((ROWS, D), jnp.float32),
        pltpu.VMEM((ROWS, D), jnp.float32),
        pltpu.VMEM((L * D,), jnp.float32),
        pltpu.VMEM((LANES,), jnp.float32),
        pltpu.VMEM((IPT,), jnp.float32),
        pltpu.VMEM((LANES, LANES), jnp.float32),
        pltpu.SemaphoreType.DMA,
        pltpu.SemaphoreType.DMA,
        pltpu.SemaphoreType.DMA,
        pltpu.SemaphoreType.DMA,
        pltpu.SemaphoreType.DMA,
        pltpu.SemaphoreType.DMA,
    ],
)
def _glove_sc(x_hbm, w_hbm, b_hbm, table_hbm, out_hbm,
              idx0, idx1, idx2, idx3, rows0, rows1, w_v, b_v, out_v, trans_v,
              sem0, sem1, isem0, isem1, isem2, isem3):
    wid = lax.axis_index("s") * NC + lax.axis_index("c")
    base = wid * IPT
    idx = (idx0, idx1, idx2, idx3)
    rows = (rows0, rows1)
    sem = (sem0, sem1)
    isem = (isem0, isem1, isem2, isem3)
    pltpu.sync_copy(w_hbm, w_v)
    pltpu.sync_copy(b_hbm, b_v)

    def idx_src(g):
        return x_hbm.at[pl.ds((base + g * K) * L, ROWS)]

    def fire_idx(g, j):
        pltpu.async_copy(idx_src(g), idx[j], isem[j])

    def wait_idx(g, j):
        pltpu.make_async_copy(idx_src(g), idx[j], isem[j]).wait()

    def fire(g, j, b):
        for off, n in STREAMS:
            pltpu.async_copy(
                table_hbm.at[idx[j].at[pl.ds(off, n)]],
                rows[b].at[pl.ds(off, n)], sem[b],
            )

    def drain(j, b):
        for off, n in STREAMS:
            pltpu.make_async_copy(
                table_hbm.at[idx[j].at[pl.ds(off, n)]],
                rows[b].at[pl.ds(off, n)], sem[b],
            ).wait()

    def compute(g, b, parity):
        rows_b = rows[b]

        def l_body(l, accs):
            new = list(accs)
            for c in range(D // LANES):
                w = w_v[pl.ds(l * D + c * LANES, LANES)]
                for k in range(K):
                    r = rows_b[k * L + l, pl.ds(c * LANES, LANES)]
                    new[k] = new[k] + r * w
            return tuple(new)

        zero = jnp.zeros((LANES,), jnp.float32)
        accs = lax.fori_loop(0, L, l_body, (zero,) * K)
        for k in range(K):
            trans_v[parity * K + k, :] = accs[k]
        if parity == 1:
            lane = lax.iota(jnp.int32, LANES)
            res = b_v[...]
            for c in range(LANES):
                col = jnp.full((LANES,), c, jnp.int32)
                res = res + plsc.load_gather(trans_v, [lane, col])
            out_v[pl.ds((g - 1) * K, LANES)] = res

    fire_idx(0, 0)
    wait_idx(0, 0)
    fire(0, 0, 0)
    fire_idx(1, 1)

    def quad_body(i, carry):
        g0 = i * 4
        for p in range(4):
            g = g0 + p

            @pl.when(g + 1 < NCH)
            def _():
                wait_idx(g + 1, (p + 1) % 4)
                fire(g + 1, (p + 1) % 4, (p + 1) % 2)

            @pl.when(g + 2 < NCH)
            def _():
                fire_idx(g + 2, (p + 2) % 4)

            drain(p % 4, p % 2)
            compute(g, p % 2, p % 2)
        return carry

    lax.fori_loop(0, NCH // 4, quad_body, 0)
    pltpu.sync_copy(out_v, out_hbm.at[pl.ds(base, IPT)])


def kernel(x, table, W, b):
    x_flat = x.reshape(B * L)
    w_flat = W.reshape(L * D)
    b16 = jnp.broadcast_to(b, (LANES,))
    return _glove_sc(x_flat, w_flat, b16, table)


# speedup vs baseline: 1.0397x; 1.0397x over previous
import functools

import jax
import jax.numpy as jnp
from jax import lax
from jax.experimental import pallas as pl
from jax.experimental.pallas import tpu as pltpu
from jax.experimental.pallas import tpu_sc as plsc

B = 16384
L = 50
V = 1000000
D = 128

NC = 2
NS = 16
NW = NC * NS
IPT = B // NW
K = 4
NCH = IPT // K
ROWS = K * L
LANES = 16
NB = 4
STREAMS = ((0, 128), (128, 72))

_mesh = plsc.VectorSubcoreMesh(core_axis_name="c", subcore_axis_name="s")


@functools.partial(
    pl.kernel,
    out_type=jax.ShapeDtypeStruct((B,), jnp.float32),
    mesh=_mesh,
    compiler_params=pltpu.CompilerParams(
        needs_layout_passes=False,
        disable_bounds_checks=True,
        disable_semaphore_checks=True,
        skip_device_barrier=True,
    ),
    scratch_types=(
        [pltpu.VMEM((ROWS,), jnp.int32) for _ in range(NB)]
        + [pltpu.VMEM((ROWS, D), jnp.float32) for _ in range(NB)]
        + [
            pltpu.VMEM((L * D,), jnp.float32),
            pltpu.VMEM((LANES,), jnp.float32),
            pltpu.VMEM((IPT,), jnp.float32),
            pltpu.VMEM((LANES, LANES), jnp.float32),
        ]
        + [pltpu.SemaphoreType.DMA for _ in range(2 * NB)]
    ),
)
def _glove_sc(x_hbm, w_hbm, b_hbm, table_hbm, out_hbm, *scratch):
    idx = scratch[:NB]
    rows = scratch[NB:2 * NB]
    w_v, b_v, out_v, trans_v = scratch[2 * NB:2 * NB + 4]
    sem = scratch[2 * NB + 4:2 * NB + 4 + NB]
    isem = scratch[2 * NB + 4 + NB:]
    wid = lax.axis_index("s") * NC + lax.axis_index("c")
    base = wid * IPT
    pltpu.sync_copy(w_hbm, w_v)
    pltpu.sync_copy(b_hbm, b_v)

    def idx_src(g):
        return x_hbm.at[pl.ds((base + g * K) * L, ROWS)]

    def fire_idx(g, j):
        pltpu.async_copy(idx_src(g), idx[j], isem[j])

    def wait_idx(g, j):
        pltpu.make_async_copy(idx_src(g), idx[j], isem[j]).wait()

    def fire(g, j):
        for off, n in STREAMS:
            pltpu.async_copy(
                table_hbm.at[idx[j].at[pl.ds(off, n)]],
                rows[j].at[pl.ds(off, n)], sem[j],
            )

    def drain(j):
        for off, n in STREAMS:
            pltpu.make_async_copy(
                table_hbm.at[idx[j].at[pl.ds(off, n)]],
                rows[j].at[pl.ds(off, n)], sem[j],
            ).wait()

    def compute(g, j, phase):
        rows_b = rows[j]

        def l_body(l, accs):
            new = list(accs)
            for c in range(D // LANES):
                w = w_v[pl.ds(l * D + c * LANES, LANES)]
                for k in range(K):
                    r = rows_b[k * L + l, pl.ds(c * LANES, LANES)]
                    new[k] = new[k] + r * w
            return tuple(new)

        zero = jnp.zeros((LANES,), jnp.float32)
        accs = lax.fori_loop(0, L, l_body, (zero,) * K)
        for k in range(K):
            trans_v[phase * K + k, :] = accs[k]
        if phase == LANES // K - 1:
            lane = lax.iota(jnp.int32, LANES)
            res = b_v[...]
            for c in range(LANES):
                col = jnp.full((LANES,), c, jnp.int32)
                res = res + plsc.load_gather(trans_v, [lane, col])
            out_v[pl.ds((g - 3) * K, LANES)] = res

    for g in range(NB - 1):
        fire_idx(g, g)
    for g in range(NB - 1):
        wait_idx(g, g)
        fire(g, g)
    fire_idx(NB - 1, NB - 1)

    def quad_body(i, carry):
        g0 = i * NB
        for p in range(NB):
            g = g0 + p

            @pl.when(g + NB - 1 < NCH)
            def _():
                wait_idx(g + NB - 1, (p + NB - 1) % NB)
                fire(g + NB - 1, (p + NB - 1) % NB)

            drain(p)

            @pl.when(g + NB < NCH)
            def _():
                fire_idx(g + NB, p)

            compute(g, p, p % (LANES // K))
        return carry

    lax.fori_loop(0, NCH // NB, quad_body, 0)
    pltpu.sync_copy(out_v, out_hbm.at[pl.ds(base, IPT)])


def kernel(x, table, W, b):
    x_flat = x.reshape(B * L)
    w_flat = W.reshape(L * D)
    b16 = jnp.broadcast_to(b, (LANES,))
    return _glove_sc(x_flat, w_flat, b16, table)
